# baseline (speedup 1.0000x reference)
import functools

import jax
import jax.numpy as jnp
from jax import lax
from jax.experimental import pallas as pl
from jax.experimental.pallas import tpu as pltpu
from jax.experimental.pallas import tpu_sc as plsc

B, H, W, T, BS, EMBED = 2, 16, 16, 12, 8, 1024
N = EMBED // 4
ROWS_PER_B = H * W * T * BS
PERIOD = T * BS
R = 32
NC, NS = 2, 16
NGROUPS = B * T
NROWS = 4 * NGROUPS * BS
RPW = NROWS // NGROUPS


def _sc_build_addend(idx, comb):
    mesh = plsc.VectorSubcoreMesh(core_axis_name="c", subcore_axis_name="s")

    @functools.partial(
        pl.kernel,
        mesh=mesh,
        out_type=jax.ShapeDtypeStruct((NROWS, N), jnp.float32),
        scratch_types=[
            pltpu.VMEM((RPW,), jnp.int32),
            pltpu.VMEM((RPW, N), jnp.float32),
            pltpu.SemaphoreType.DMA,
        ],
    )
    def build(idx_hbm, comb_hbm, out_hbm, idx_v, rows_v, sem):
        wid = lax.axis_index("s") * NC + lax.axis_index("c")

        @pl.when(wid < NGROUPS)
        def _():
            base = wid * RPW
            pltpu.sync_copy(idx_hbm.at[pl.ds(base, RPW)], idx_v)
            pltpu.async_copy(comb_hbm.at[idx_v], rows_v, sem).wait()
            pltpu.sync_copy(rows_v, out_hbm.at[pl.ds(base, RPW)])

    return build(idx, comb)


def _tc_body(tokens_ref,
             addend_ref,
             out_ref):
    out_ref[...] = tokens_ref[...] + addend_ref[0][None, :, :]


@jax.jit
def kernel(modality_tokens, timestamps, channel_embed, pos_embed, month_table):
    comb = jnp.concatenate(
        [channel_embed, pos_embed[:T], month_table,
         jnp.zeros((1, N), jnp.float32)], axis=0)
    months = timestamps[:, :, 1].reshape(-1).astype(jnp.int32)
    mon_idx = jnp.repeat(months, BS) + (BS + T)
    ch_idx = jnp.tile(jnp.arange(BS, dtype=jnp.int32), NGROUPS)
    pos_idx = jnp.repeat(jnp.tile(jnp.arange(T, dtype=jnp.int32), B),
                         BS) + BS
    zero_idx = jnp.full((NGROUPS * BS,), BS + 2 * T, jnp.int32)
    idx = jnp.stack([ch_idx, pos_idx, mon_idx, zero_idx],
                    axis=1).reshape(-1)

    addend = _sc_build_addend(idx, comb).reshape(B, PERIOD, EMBED)

    tokens = modality_tokens.reshape(-1, PERIOD, EMBED)
    num_blocks = tokens.shape[0] // R
    steps_per_b = ROWS_PER_B // (R * PERIOD)

    out = pl.pallas_call(
        _tc_body,
        grid=(num_blocks,),
        in_specs=[
            pl.BlockSpec((R, PERIOD, EMBED), lambda i: (i, 0, 0)),
            pl.BlockSpec((1, PERIOD, EMBED),
                         lambda i: (i // steps_per_b, 0, 0)),
        ],
        out_specs=pl.BlockSpec((R, PERIOD, EMBED), lambda i: (i, 0, 0)),
        out_shape=jax.ShapeDtypeStruct(tokens.shape, jnp.float32),
    )(tokens, addend)
    return out.reshape(B, H, W, T, BS, EMBED)


# speedup vs baseline: 1.0457x; 1.0457x over previous
import functools

import jax
import jax.numpy as jnp
from jax import lax
from jax.experimental import pallas as pl
from jax.experimental.pallas import tpu as pltpu
from jax.experimental.pallas import tpu_sc as plsc

B, H, W, T, BS, EMBED = 2, 16, 16, 12, 8, 1024
N = EMBED // 4
ROWS_PER_B = H * W * T * BS
PERIOD = T * BS
R = 32
NC, NS = 2, 16
GROUP = BS
NGROUPS = B * T


NROWS = 3 * NGROUPS * GROUP
RPW = NROWS // NGROUPS


def _sc_build_addend(idx, comb):
    mesh = plsc.VectorSubcoreMesh(core_axis_name="c", subcore_axis_name="s")

    @functools.partial(
        pl.kernel,
        mesh=mesh,
        out_type=jax.ShapeDtypeStruct((NROWS, N), jnp.float32),
        scratch_types=[
            pltpu.VMEM((RPW,), jnp.int32),
            pltpu.VMEM((RPW, N), jnp.float32),
            pltpu.SemaphoreType.DMA,
        ],
    )
    def build(idx_hbm, comb_hbm, out_hbm, idx_v, rows_v, sem):
        wid = lax.axis_index("s") * NC + lax.axis_index("c")

        @pl.when(wid < NGROUPS)
        def _():
            base = wid * RPW
            pltpu.sync_copy(idx_hbm.at[pl.ds(base, RPW)], idx_v)
            pltpu.async_copy(comb_hbm.at[idx_v], rows_v, sem).wait()
            pltpu.sync_copy(rows_v, out_hbm.at[pl.ds(base, RPW)])

    return build(idx, comb)


def _tc_body(tokens_ref,
             addend_ref,
             out_ref):
    add = addend_ref[0]
    out_ref[:, :, 0:3 * N] = tokens_ref[:, :, 0:3 * N] + add[None, :, :]
    out_ref[:, :, 3 * N:] = tokens_ref[:, :, 3 * N:]


@jax.jit
def kernel(modality_tokens, timestamps, channel_embed, pos_embed, month_table):
    comb = jnp.concatenate(
        [channel_embed, pos_embed[:T], month_table], axis=0)
    months = timestamps[:, :, 1].reshape(-1).astype(jnp.int32)
    mon_idx = jnp.repeat(months, GROUP) + (BS + T)
    ch_idx = jnp.tile(jnp.arange(BS, dtype=jnp.int32), NGROUPS)
    pos_idx = jnp.repeat(jnp.tile(jnp.arange(T, dtype=jnp.int32), B),
                         GROUP) + BS
    idx = jnp.stack([ch_idx, pos_idx, mon_idx], axis=1).reshape(-1)

    addend = _sc_build_addend(idx, comb)
    addend = addend.reshape(B, PERIOD, 3 * N)

    tokens = modality_tokens.reshape(-1, PERIOD, EMBED)
    num_blocks = tokens.shape[0] // R
    steps_per_b = ROWS_PER_B // (R * PERIOD)

    out = pl.pallas_call(
        _tc_body,
        grid=(num_blocks,),
        in_specs=[
            pl.BlockSpec((R, PERIOD, EMBED), lambda i: (i, 0, 0)),
            pl.BlockSpec((1, PERIOD, 3 * N),
                         lambda i: (i // steps_per_b, 0, 0)),
        ],
        out_specs=pl.BlockSpec((R, PERIOD, EMBED), lambda i: (i, 0, 0)),
        out_shape=jax.ShapeDtypeStruct(tokens.shape, jnp.float32),
    )(tokens, addend)
    return out.reshape(B, H, W, T, BS, EMBED)


# speedup vs baseline: 1.0664x; 1.0198x over previous
import functools

import jax
import jax.numpy as jnp
from jax import lax
from jax.experimental import pallas as pl
from jax.experimental.pallas import tpu as pltpu
from jax.experimental.pallas import tpu_sc as plsc

B, H, W, T, BS, EMBED = 2, 16, 16, 12, 8, 1024
N = EMBED // 4
ROWS_PER_B = H * W * T * BS
PERIOD = T * BS
R = 32
STEPS_PER_B = ROWS_PER_B // (R * PERIOD)
NC, NS = 2, 16
NGROUPS = T
NROWS = 3 * NGROUPS * BS
RPW = NROWS // NGROUPS


def _sc_build_addend(idx, comb):
    mesh = plsc.VectorSubcoreMesh(core_axis_name="c", subcore_axis_name="s")

    @functools.partial(
        pl.kernel,
        mesh=mesh,
        out_type=jax.ShapeDtypeStruct((NROWS, N), jnp.float32),
        scratch_types=[
            pltpu.VMEM((RPW,), jnp.int32),
            pltpu.VMEM((RPW, N), jnp.float32),
            pltpu.SemaphoreType.DMA,
        ],
    )
    def build(idx_hbm, comb_hbm, out_hbm, idx_v, rows_v, sem):
        wid = lax.axis_index("s") * NC + lax.axis_index("c")

        @pl.when(wid < NGROUPS)
        def _():
            base = wid * RPW
            pltpu.sync_copy(idx_hbm.at[pl.ds(base, RPW)], idx_v)
            pltpu.async_copy(comb_hbm.at[idx_v], rows_v, sem).wait()
            pltpu.sync_copy(rows_v, out_hbm.at[pl.ds(base, RPW)])

    return build(idx, comb)


def _tc1_body(months_ref,
              tokens_ref,
              channel_ref,
              pos_ref,
              month_ref,
              out_ref,
              addend_ref):
    i = pl.program_id(0)

    @pl.when(i == 0)
    def _build_addend():
        for t in range(T):
            row0 = t * BS
            addend_ref[pl.ds(row0, BS), 0:N] = channel_ref[...]
            addend_ref[pl.ds(row0, BS), N:2 * N] = jnp.broadcast_to(
                pos_ref[t, :][None, :], (BS, N))
            m = months_ref[t]
            addend_ref[pl.ds(row0, BS), 2 * N:3 * N] = jnp.broadcast_to(
                month_ref[m, :][None, :], (BS, N))
            addend_ref[pl.ds(row0, BS), 3 * N:] = jnp.zeros((BS, N),
                                                            jnp.float32)

    out_ref[...] = tokens_ref[...] + addend_ref[...][None, :, :]


def _tc2_body(prev_ref,
              tokens_ref,
              addend_ref,
              out_ref):
    del prev_ref
    add = addend_ref[0]
    out_ref[:, :, 0:3 * N] = tokens_ref[:, :, 0:3 * N] + add[None, :, :]
    out_ref[:, :, 3 * N:] = tokens_ref[:, :, 3 * N:]


@jax.jit
def kernel(modality_tokens, timestamps, channel_embed, pos_embed, month_table):
    months = timestamps[:, :, 1].astype(jnp.int32)

    comb = jnp.concatenate(
        [channel_embed, pos_embed[:T], month_table], axis=0)
    mon_idx = jnp.repeat(months[1], BS) + (BS + T)
    ch_idx = jnp.tile(jnp.arange(BS, dtype=jnp.int32), T)
    pos_idx = jnp.repeat(jnp.arange(T, dtype=jnp.int32), BS) + BS
    idx = jnp.stack([ch_idx, pos_idx, mon_idx], axis=1).reshape(-1)

    tokens = modality_tokens.reshape(-1, PERIOD, EMBED)

    grid_spec = pltpu.PrefetchScalarGridSpec(
        num_scalar_prefetch=1,
        grid=(STEPS_PER_B,),
        in_specs=[
            pl.BlockSpec((R, PERIOD, EMBED), lambda i, m: (i, 0, 0)),
            pl.BlockSpec((BS, N), lambda i, m: (0, 0)),
            pl.BlockSpec((T, N), lambda i, m: (0, 0)),
            pl.BlockSpec((12, N), lambda i, m: (0, 0)),
        ],
        out_specs=pl.BlockSpec((R, PERIOD, EMBED), lambda i, m: (i, 0, 0)),
        scratch_shapes=[pltpu.VMEM((PERIOD, EMBED), jnp.float32)],
    )
    half_out = pl.pallas_call(
        _tc1_body,
        grid_spec=grid_spec,
        out_shape=jax.ShapeDtypeStruct(tokens.shape, jnp.float32),
    )(months[0], tokens, channel_embed, pos_embed[:T], month_table)

    addend1 = _sc_build_addend(idx, comb).reshape(1, PERIOD, 3 * N)

    out = pl.pallas_call(
        _tc2_body,
        grid=(STEPS_PER_B,),
        in_specs=[
            pl.BlockSpec(memory_space=pl.ANY),
            pl.BlockSpec((R, PERIOD, EMBED),
                         lambda i: (i + STEPS_PER_B, 0, 0)),
            pl.BlockSpec((1, PERIOD, 3 * N), lambda i: (0, 0, 0)),
        ],
        out_specs=pl.BlockSpec((R, PERIOD, EMBED),
                               lambda i: (i + STEPS_PER_B, 0, 0)),
        out_shape=jax.ShapeDtypeStruct(tokens.shape, jnp.float32),
        input_output_aliases={0: 0},
    )(half_out, tokens, addend1)
    return out.reshape(B, H, W, T, BS, EMBED)


# speedup vs baseline: 1.0724x; 1.0056x over previous
import functools

import jax
import jax.numpy as jnp
from jax import lax
from jax.experimental import pallas as pl
from jax.experimental.pallas import tpu as pltpu
from jax.experimental.pallas import tpu_sc as plsc

B, H, W, T, BS, EMBED = 2, 16, 16, 12, 8, 1024
N = EMBED // 4
ROWS_PER_B = H * W * T * BS
PERIOD = T * BS
R = 32
STEPS_PER_B = ROWS_PER_B // (R * PERIOD)
NC, NS = 2, 16
NGROUPS = T
NROWS = 3 * NGROUPS * BS
RPW = NROWS // NGROUPS


def _sc_build_addend(idx, comb):
    mesh = plsc.VectorSubcoreMesh(core_axis_name="c", subcore_axis_name="s")

    @functools.partial(
        pl.kernel,
        mesh=mesh,
        out_type=jax.ShapeDtypeStruct((NROWS, N), jnp.float32),
        scratch_types=[
            pltpu.VMEM((RPW,), jnp.int32),
            pltpu.VMEM((RPW, N), jnp.float32),
            pltpu.SemaphoreType.DMA,
        ],
    )
    def build(idx_hbm, comb_hbm, out_hbm, idx_v, rows_v, sem):
        wid = lax.axis_index("s") * NC + lax.axis_index("c")

        @pl.when(wid < NGROUPS)
        def _():
            base = wid * RPW
            pltpu.sync_copy(idx_hbm.at[pl.ds(base, RPW)], idx_v)
            pltpu.async_copy(comb_hbm.at[idx_v], rows_v, sem).wait()
            pltpu.sync_copy(rows_v, out_hbm.at[pl.ds(base, RPW)])

    return build(idx, comb)


TC1_STEPS = 12
TC2_STEPS = 2 * STEPS_PER_B - TC1_STEPS


def _tc1_body(months_ref,
              tokens_ref,
              channel_ref,
              pos_ref,
              month_ref,
              out_ref,
              addend_ref):
    i = pl.program_id(0)

    @pl.when(i == 0)
    def _build_addend():
        for b in range(B):
            for t in range(T):
                row0 = b * PERIOD + t * BS
                addend_ref[pl.ds(row0, BS), 0:N] = channel_ref[...]
                addend_ref[pl.ds(row0, BS), N:2 * N] = jnp.broadcast_to(
                    pos_ref[t, :][None, :], (BS, N))
                m = months_ref[b * T + t]
                addend_ref[pl.ds(row0, BS), 2 * N:3 * N] = jnp.broadcast_to(
                    month_ref[m, :][None, :], (BS, N))
                addend_ref[pl.ds(row0, BS), 3 * N:] = jnp.zeros(
                    (BS, N), jnp.float32)

    b = i // STEPS_PER_B
    add = addend_ref[pl.ds(b * PERIOD, PERIOD), :]
    out_ref[...] = tokens_ref[...] + add[None, :, :]


def _tc2_body(prev_ref,
              tokens_ref,
              addend_ref,
              out_ref):
    del prev_ref
    add = addend_ref[0]
    out_ref[:, :, 0:3 * N] = tokens_ref[:, :, 0:3 * N] + add[None, :, :]
    out_ref[:, :, 3 * N:] = tokens_ref[:, :, 3 * N:]


@jax.jit
def kernel(modality_tokens, timestamps, channel_embed, pos_embed, month_table):
    months = timestamps[:, :, 1].astype(jnp.int32)

    comb = jnp.concatenate(
        [channel_embed, pos_embed[:T], month_table], axis=0)
    mon_idx = jnp.repeat(months[1], BS) + (BS + T)
    ch_idx = jnp.tile(jnp.arange(BS, dtype=jnp.int32), T)
    pos_idx = jnp.repeat(jnp.arange(T, dtype=jnp.int32), BS) + BS
    idx = jnp.stack([ch_idx, pos_idx, mon_idx], axis=1).reshape(-1)

    tokens = modality_tokens.reshape(-1, PERIOD, EMBED)

    grid_spec = pltpu.PrefetchScalarGridSpec(
        num_scalar_prefetch=1,
        grid=(TC1_STEPS,),
        in_specs=[
            pl.BlockSpec((R, PERIOD, EMBED), lambda i, m: (i, 0, 0)),
            pl.BlockSpec((BS, N), lambda i, m: (0, 0)),
            pl.BlockSpec((T, N), lambda i, m: (0, 0)),
            pl.BlockSpec((12, N), lambda i, m: (0, 0)),
        ],
        out_specs=pl.BlockSpec((R, PERIOD, EMBED), lambda i, m: (i, 0, 0)),
        scratch_shapes=[pltpu.VMEM((B * PERIOD, EMBED), jnp.float32)],
    )
    half_out = pl.pallas_call(
        _tc1_body,
        grid_spec=grid_spec,
        out_shape=jax.ShapeDtypeStruct(tokens.shape, jnp.float32),
    )(months.reshape(-1), tokens, channel_embed, pos_embed[:T], month_table)

    addend1 = _sc_build_addend(idx, comb).reshape(1, PERIOD, 3 * N)

    out = pl.pallas_call(
        _tc2_body,
        grid=(TC2_STEPS,),
        in_specs=[
            pl.BlockSpec(memory_space=pl.ANY),
            pl.BlockSpec((R, PERIOD, EMBED),
                         lambda i: (i + TC1_STEPS, 0, 0)),
            pl.BlockSpec((1, PERIOD, 3 * N), lambda i: (0, 0, 0)),
        ],
        out_specs=pl.BlockSpec((R, PERIOD, EMBED),
                               lambda i: (i + TC1_STEPS, 0, 0)),
        out_shape=jax.ShapeDtypeStruct(tokens.shape, jnp.float32),
        input_output_aliases={0: 0},
    )(half_out, tokens, addend1)
    return out.reshape(B, H, W, T, BS, EMBED)
